# baseline (speedup 1.0000x reference)
import functools

import jax
import jax.numpy as jnp
from jax import lax
from jax.experimental import pallas as pl
from jax.experimental.pallas import tpu as pltpu
from jax.experimental.pallas import tpu_sc as plsc

NB = 512
SCALE = NB / 16.0
K0 = NB // 2
ROW = 4 * NB
NH = 2
EPS = 1e-6

NC, NS, L = 2, 16, 16
NW = NC * NS

S = 7
B = 8
P = 512 * 512
QS = 4
TASK = P // QS
CH = 8192


def _phase1(logits1, target1):
    mesh = plsc.VectorSubcoreMesh(
        core_axis_name="c", subcore_axis_name="s",
        num_cores=NC, num_subcores=NS)

    nchunk = TASK // CH

    @functools.partial(
        pl.kernel,
        out_type=jax.ShapeDtypeStruct((NW * NH * S * ROW,), jnp.float32),
        mesh=mesh,
        scratch_types=[
            pltpu.VMEM((TASK,), jnp.float32),
            pltpu.VMEM((CH,), jnp.float32),
            pltpu.VMEM((CH,), jnp.float32),
            pltpu.VMEM((NH * S * ROW,), jnp.float32),
            pltpu.SemaphoreType.DMA,
            pltpu.SemaphoreType.DMA,
            pltpu.SemaphoreType.DMA,
        ],
        compiler_params=pltpu.CompilerParams(needs_layout_passes=False),
    )
    def k(log_hbm, tgt_hbm, parts_hbm, tgt_v, buf_a, buf_b, hist,
          sem_t, sem_a, sem_b):
        wid = lax.axis_index("s") * NC + lax.axis_index("c")
        b = wid // QS
        q = wid % QS

        def log_off(s):
            return (s * B + b) * P + q * TASK

        pltpu.async_copy(tgt_hbm.at[pl.ds(b * P + q * TASK, TASK)],
                         tgt_v, sem_t)
        pltpu.async_copy(log_hbm.at[pl.ds(log_off(0), CH)], buf_a, sem_a)

        zero = jnp.zeros((L,), jnp.float32)
        ones = jnp.ones((L,), jnp.float32)

        @plsc.parallel_loop(0, NH * S * ROW, L)
        def _zero_body(kk):
            hist[pl.ds(kk, L)] = zero

        pltpu.make_async_copy(tgt_hbm.at[pl.ds(0, TASK)], tgt_v, sem_t).wait()

        def process(buf, c, s):
            t_base = c * CH

            @plsc.parallel_loop(0, CH, NH * L, unroll=4)
            def _body(i):
                for h in range(NH):
                    base = h * (S * ROW) + s * ROW
                    bias = float(K0 + NB + base)
                    lo = float(base + NB)
                    hi = float(base + 2 * NB - 1)
                    x = buf[pl.ds(i + h * L, L)]
                    t = tgt_v[pl.ds(t_base + i + h * L, L)]
                    e = 1.0 - x * (2.0 * t - 1.0)
                    a = jnp.minimum(jnp.maximum(e * SCALE + bias, lo), hi)
                    ic = (a - t * float(NB)).astype(jnp.int32)
                    plsc.addupdate_scatter(hist, [ic], ones)
                    plsc.addupdate_scatter(hist, [ic + 2 * NB], e)

        for s in range(S):
            def body2(c2, carry, s=s):
                c_even = c2 * 2
                pltpu.make_async_copy(
                    log_hbm.at[pl.ds(0, CH)], buf_a, sem_a).wait()
                pltpu.async_copy(
                    log_hbm.at[pl.ds(log_off(s) + (c_even + 1) * CH, CH)],
                    buf_b, sem_b)
                process(buf_a, c_even, s)
                pltpu.make_async_copy(
                    log_hbm.at[pl.ds(0, CH)], buf_b, sem_b).wait()
                nxt_same = log_off(s) + (c_even + 2) * CH
                nxt_s = log_off(s + 1) if s < S - 1 else log_off(s)
                nxt = jnp.where(c_even + 2 >= nchunk, nxt_s, nxt_same)
                pltpu.async_copy(log_hbm.at[pl.ds(nxt, CH)], buf_a, sem_a)
                process(buf_b, c_even + 1, s)
                return carry

            lax.fori_loop(0, nchunk // 2, body2, 0)

        pltpu.make_async_copy(log_hbm.at[pl.ds(0, CH)], buf_a, sem_a).wait()

        pltpu.sync_copy(
            hist, parts_hbm.at[pl.ds(wid * (NH * S * ROW), NH * S * ROW)])

    return k(logits1, target1)


def _cumsum_last(x):
    n = x.shape[-1]
    sh = 1
    while sh < n:
        shifted = jnp.concatenate(
            [jnp.zeros_like(x[..., :sh]), x[..., :-sh]], axis=-1)
        x = x + shifted
        sh *= 2
    return x


def _finalize(parts):
    def body(p_ref, out_ref):
        xs = jnp.sum(p_ref[...], axis=1)
        Hp = xs[..., 0 * NB:1 * NB]
        Hn = xs[..., 1 * NB:2 * NB]
        Ep = xs[..., 2 * NB:3 * NB]
        En = xs[..., 3 * NB:4 * NB]
        g = jnp.sum(Hp[:, 0, :], axis=-1)[:, None]
        gb = g[:, :, None]
        vmask = (lax.broadcasted_iota(jnp.int32, (1, 1, NB), 2) >= K0
                 ).astype(jnp.float32)
        Epv = Ep * vmask
        Env = En * vmask
        cn = _cumsum_last(Hn)
        cp = _cumsum_last(Hp)
        SAn = cn[..., NB - 1:NB] - cn
        SAp = cp[..., NB - 1:NB] - cp
        inv0 = 1.0 / (gb + SAn + EPS)
        s_pos = jnp.sum(Epv * inv0, axis=-1)
        d = (inv0 - 1.0 / (gb + SAn + Hn + EPS)) / jnp.maximum(Hn, 1.0)
        s_neg = jnp.sum(Env * (gb - SAp - Hp) * d, axis=-1)
        lh = jnp.mean(s_pos + s_neg, axis=0)
        ep = Ep / jnp.maximum(Hp, 1.0)
        en = En / jnp.maximum(Hn, 1.0)
        pp = jnp.sum(Hp * jax.nn.sigmoid(1.0 - ep), axis=-1)
        pn = jnp.sum(Hn * jax.nn.sigmoid(en - 1.0), axis=-1)
        inter = jnp.sum(pp, axis=0)
        probs = inter + jnp.sum(pn, axis=0)
        tsum = jnp.sum(g)
        dl = 1.0 - (2.0 * inter + 1.0) / (probs + tsum + 1.0)
        comb = lh + dl
        comb = jnp.where(jnp.isnan(comb) | jnp.isinf(comb), 0.0, comb)
        out_ref[...] = (jnp.sum(comb) + comb[0]).reshape(1, 1)

    return pl.pallas_call(
        body, out_shape=jax.ShapeDtypeStruct((1, 1), jnp.float32))(parts)


def kernel(outputs, target):
    logits1 = outputs.astype(jnp.float32).reshape(S * B * P)
    target1 = target.astype(jnp.float32).reshape(B * P)
    parts = _phase1(logits1, target1)
    total = _finalize(parts.reshape(B, QS * NH, S, ROW))
    return total[0, 0]


# speedup vs baseline: 1.0133x; 1.0133x over previous
import functools

import jax
import jax.numpy as jnp
from jax import lax
from jax.experimental import pallas as pl
from jax.experimental.pallas import tpu as pltpu
from jax.experimental.pallas import tpu_sc as plsc

NB = 1024
SCALE = NB / 16.0
K0 = NB // 2
ROW = 4 * NB
NH = 1
EPS = 1e-6

NC, NS, L = 2, 16, 16
NW = NC * NS

S = 7
B = 8
P = 512 * 512
QS = 4
TASK = P // QS
CH = 8192


def _phase1(logits1, target1):
    mesh = plsc.VectorSubcoreMesh(
        core_axis_name="c", subcore_axis_name="s",
        num_cores=NC, num_subcores=NS)

    nchunk = TASK // CH

    @functools.partial(
        pl.kernel,
        out_type=jax.ShapeDtypeStruct((NW * NH * S * ROW,), jnp.float32),
        mesh=mesh,
        scratch_types=[
            pltpu.VMEM((TASK,), jnp.float32),
            pltpu.VMEM((CH,), jnp.float32),
            pltpu.VMEM((CH,), jnp.float32),
            pltpu.VMEM((NH * S * ROW,), jnp.float32),
            pltpu.SemaphoreType.DMA,
            pltpu.SemaphoreType.DMA,
            pltpu.SemaphoreType.DMA,
        ],
        compiler_params=pltpu.CompilerParams(
            needs_layout_passes=False, use_tc_tiling_on_sc=False),
    )
    def k(log_hbm, tgt_hbm, parts_hbm, tgt_v, buf_a, buf_b, hist,
          sem_t, sem_a, sem_b):
        wid = lax.axis_index("s") * NC + lax.axis_index("c")
        b = wid // QS
        q = wid % QS

        def log_off(s):
            return (s * B + b) * P + q * TASK

        pltpu.async_copy(tgt_hbm.at[pl.ds(b * P + q * TASK, TASK)],
                         tgt_v, sem_t)
        pltpu.async_copy(log_hbm.at[pl.ds(log_off(0), CH)], buf_a, sem_a)

        zero = jnp.zeros((L,), jnp.float32)
        ones = jnp.ones((L,), jnp.float32)

        @plsc.parallel_loop(0, NH * S * ROW, L)
        def _zero_body(kk):
            hist[pl.ds(kk, L)] = zero

        pltpu.make_async_copy(tgt_hbm.at[pl.ds(0, TASK)], tgt_v, sem_t).wait()

        def process(buf, c, s):
            t_base = c * CH

            @plsc.parallel_loop(0, CH, NH * L, unroll=4)
            def _body(i):
                for h in range(NH):
                    base = h * (S * ROW) + s * ROW
                    bias = float(K0 + NB + base)
                    lo = float(base + NB)
                    hi = float(base + 2 * NB - 1)
                    x = buf[pl.ds(i + h * L, L)]
                    t = tgt_v[pl.ds(t_base + i + h * L, L)]
                    e = 1.0 - x * (2.0 * t - 1.0)
                    a = jnp.minimum(jnp.maximum(e * SCALE + bias, lo), hi)
                    ic = (a - t * float(NB)).astype(jnp.int32)
                    plsc.addupdate_scatter(hist, [ic], ones)
                    plsc.addupdate_scatter(hist, [ic + 2 * NB], e)

        for s in range(S):
            def body2(c2, carry, s=s):
                c_even = c2 * 2
                pltpu.make_async_copy(
                    log_hbm.at[pl.ds(0, CH)], buf_a, sem_a).wait()
                pltpu.async_copy(
                    log_hbm.at[pl.ds(log_off(s) + (c_even + 1) * CH, CH)],
                    buf_b, sem_b)
                process(buf_a, c_even, s)
                pltpu.make_async_copy(
                    log_hbm.at[pl.ds(0, CH)], buf_b, sem_b).wait()
                nxt_same = log_off(s) + (c_even + 2) * CH
                nxt_s = log_off(s + 1) if s < S - 1 else log_off(s)
                nxt = jnp.where(c_even + 2 >= nchunk, nxt_s, nxt_same)
                pltpu.async_copy(log_hbm.at[pl.ds(nxt, CH)], buf_a, sem_a)
                process(buf_b, c_even + 1, s)
                return carry

            lax.fori_loop(0, nchunk // 2, body2, 0)

        pltpu.make_async_copy(log_hbm.at[pl.ds(0, CH)], buf_a, sem_a).wait()

        pltpu.sync_copy(
            hist, parts_hbm.at[pl.ds(wid * (NH * S * ROW), NH * S * ROW)])

    return k(logits1, target1)


def _cumsum_last(x):
    n = x.shape[-1]
    sh = 1
    while sh < n:
        shifted = jnp.concatenate(
            [jnp.zeros_like(x[..., :sh]), x[..., :-sh]], axis=-1)
        x = x + shifted
        sh *= 2
    return x


def _finalize(parts):
    def body(p_ref, out_ref):
        xs = jnp.sum(p_ref[...], axis=1)
        Hp = xs[..., 0 * NB:1 * NB]
        Hn = xs[..., 1 * NB:2 * NB]
        Ep = xs[..., 2 * NB:3 * NB]
        En = xs[..., 3 * NB:4 * NB]
        g = jnp.sum(Hp[:, 0, :], axis=-1)[:, None]
        gb = g[:, :, None]
        vmask = (lax.broadcasted_iota(jnp.int32, (1, 1, NB), 2) >= K0
                 ).astype(jnp.float32)
        Epv = Ep * vmask
        Env = En * vmask
        cn = _cumsum_last(Hn)
        cp = _cumsum_last(Hp)
        SAn = cn[..., NB - 1:NB] - cn
        SAp = cp[..., NB - 1:NB] - cp
        inv0 = 1.0 / (gb + SAn + EPS)
        s_pos = jnp.sum(Epv * inv0, axis=-1)
        d = (inv0 - 1.0 / (gb + SAn + Hn + EPS)) / jnp.maximum(Hn, 1.0)
        s_neg = jnp.sum(Env * (gb - SAp - Hp) * d, axis=-1)
        lh = jnp.mean(s_pos + s_neg, axis=0)
        ep = Ep / jnp.maximum(Hp, 1.0)
        en = En / jnp.maximum(Hn, 1.0)
        pp = jnp.sum(Hp * jax.nn.sigmoid(1.0 - ep), axis=-1)
        pn = jnp.sum(Hn * jax.nn.sigmoid(en - 1.0), axis=-1)
        inter = jnp.sum(pp, axis=0)
        probs = inter + jnp.sum(pn, axis=0)
        tsum = jnp.sum(g)
        dl = 1.0 - (2.0 * inter + 1.0) / (probs + tsum + 1.0)
        comb = lh + dl
        comb = jnp.where(jnp.isnan(comb) | jnp.isinf(comb), 0.0, comb)
        out_ref[...] = (jnp.sum(comb) + comb[0]).reshape(1, 1)

    return pl.pallas_call(
        body, out_shape=jax.ShapeDtypeStruct((1, 1), jnp.float32))(parts)


def kernel(outputs, target):
    logits1 = outputs.astype(jnp.float32).reshape(S * B * P)
    target1 = target.astype(jnp.float32).reshape(B * P)
    parts = _phase1(logits1, target1)
    total = _finalize(parts.reshape(B, QS * NH, S, ROW))
    return total[0, 0]
